# Initial kernel scaffold; baseline (speedup 1.0000x reference)
import jax
import jax.numpy as jnp
from jax.experimental import pallas as pl


def kernel(x, gate_w, w_gate, w_up, w_down, mlp_w1, mlp_w3, mlp_w2, shared_gate_w):
    raise NotImplementedError("write your pallas kernel here")


# speedup vs baseline: 11.4570x; 11.4570x over previous
import jax
import jax.numpy as jnp
from jax import lax
from jax.experimental import pallas as pl
from jax.experimental.pallas import tpu as pltpu


def _dotT(a, b):
    return lax.dot_general(a, b, (((1,), (1,)), ((), ())),
                           preferred_element_type=jnp.float32)


def _routed_kernel(x_ref, gate_w_ref, wg_ref, wu_ref, wd_ref,
                   out_ref, mask_ref):
    e = pl.program_id(0)
    x = x_ref[...]

    @pl.when(e == 0)
    def _init():
        scores = _dotT(x, gate_w_ref[...])
        n, n_exp = scores.shape
        idx = lax.broadcasted_iota(jnp.int32, (n, n_exp), 1)
        m1 = jnp.max(scores, axis=1, keepdims=True)
        a1 = jnp.min(jnp.where(scores == m1, idx, n_exp), axis=1, keepdims=True)
        sel1 = idx == a1
        scores2 = jnp.where(sel1, jnp.float32(-jnp.inf), scores)
        m2 = jnp.max(scores2, axis=1, keepdims=True)
        a2 = jnp.min(jnp.where(scores2 == m2, idx, n_exp), axis=1, keepdims=True)
        sel2 = idx == a2
        w1 = jax.nn.sigmoid(m1 - m2)
        mask_ref[...] = (w1 * sel1.astype(jnp.float32)
                         + (1.0 - w1) * sel2.astype(jnp.float32))
        out_ref[...] = jnp.zeros_like(out_ref)

    mask = mask_ref[...]
    col = lax.broadcasted_iota(jnp.int32, mask.shape, 1) == e
    me = jnp.sum(jnp.where(col, mask, 0.0), axis=1, keepdims=True)
    g = _dotT(x, wg_ref[0])
    u = _dotT(x, wu_ref[0])
    h = jax.nn.silu(g) * u * me
    out_ref[...] += _dotT(h, wd_ref[0])


def _shared_kernel(x_ref, sgw_ref, routed_ref, w1_ref, w3_ref, w2_ref,
                   out_ref, sg_ref):
    j = pl.program_id(0)
    x = x_ref[...]

    @pl.when(j == 0)
    def _init():
        sg_ref[...] = jax.nn.sigmoid(_dotT(x, sgw_ref[...]))
        out_ref[...] = routed_ref[...]

    s1 = _dotT(x, w1_ref[...])
    s3 = _dotT(x, w3_ref[...])
    sh = jax.nn.silu(s1) * s3
    out_ref[...] += sg_ref[...] * _dotT(sh, w2_ref[...])


def kernel(x, gate_w, w_gate, w_up, w_down, mlp_w1, mlp_w3, mlp_w2, shared_gate_w):
    B, T, D = x.shape
    E, H, _ = w_gate.shape
    HS = mlp_w1.shape[0]
    N = B * T
    x_flat = x.reshape(N, D)

    routed = pl.pallas_call(
        _routed_kernel,
        grid=(E,),
        in_specs=[
            pl.BlockSpec((N, D), lambda e: (0, 0)),
            pl.BlockSpec((E, D), lambda e: (0, 0)),
            pl.BlockSpec((1, H, D), lambda e: (e, 0, 0)),
            pl.BlockSpec((1, H, D), lambda e: (e, 0, 0)),
            pl.BlockSpec((1, D, H), lambda e: (e, 0, 0)),
        ],
        out_specs=pl.BlockSpec((N, D), lambda e: (0, 0)),
        out_shape=jax.ShapeDtypeStruct((N, D), jnp.float32),
        scratch_shapes=[pltpu.VMEM((N, E), jnp.float32)],
    )(x_flat, gate_w, w_gate, w_up, w_down)

    n_s = 11
    HSc = HS // n_s
    out = pl.pallas_call(
        _shared_kernel,
        grid=(n_s,),
        in_specs=[
            pl.BlockSpec((N, D), lambda j: (0, 0)),
            pl.BlockSpec((1, D), lambda j: (0, 0)),
            pl.BlockSpec((N, D), lambda j: (0, 0)),
            pl.BlockSpec((HSc, D), lambda j: (j, 0)),
            pl.BlockSpec((HSc, D), lambda j: (j, 0)),
            pl.BlockSpec((D, HSc), lambda j: (0, j)),
        ],
        out_specs=pl.BlockSpec((N, D), lambda j: (0, 0)),
        out_shape=jax.ShapeDtypeStruct((N, D), jnp.float32),
        scratch_shapes=[pltpu.VMEM((N, 1), jnp.float32)],
    )(x_flat, shared_gate_w, routed, mlp_w1, mlp_w3, mlp_w2)
    return out.reshape(B, T, D)
